# baseline (speedup 1.0000x reference)
import dataclasses
import functools

import jax
import jax.numpy as jnp
from jax import lax
from jax.experimental import pallas as pl
from jax.experimental.pallas import tpu as pltpu
from jax.experimental.pallas import tpu_sc as plsc

_B, _C, _H, _W = 4, 128, 50, 50
_D = 128
_MH, _HD = 8, 16
_N = _B * _H * _W
_E = 320000
_NW = 32
_K = 64
_P = 32
_NB = 4
_CH0 = 160
_CH1 = 160
_TOTCH = 16 * (_CH0 + _CH1)
_EPW = 10240
_EPAD = _TOTCH * _K
_NPAD = 10240

_mesh = plsc.VectorSubcoreMesh(core_axis_name="c", subcore_axis_name="s")

_sc_params = pltpu.CompilerParams()
if "needs_layout_passes" in pltpu.CompilerParams.__dataclass_fields__:
    _sc_params = dataclasses.replace(_sc_params, needs_layout_passes=False)


@functools.partial(
    pl.kernel,
    out_type=jax.ShapeDtypeStruct((_NW, _NPAD), jnp.float32),
    mesh=_mesh,
    compiler_params=_sc_params,
    scratch_types=[
        pltpu.VMEM((_EPW,), jnp.int32),
        pltpu.VMEM((_NPAD,), jnp.float32),
    ],
)
def _deg_kernel(dst_flat, z1d, degp, dstv, degv):
    cid = lax.axis_index("c")
    sid = lax.axis_index("s")
    wid = sid * 2 + cid
    pltpu.sync_copy(z1d, degv)
    pltpu.sync_copy(dst_flat.at[wid], dstv)
    ones = jnp.full((16,), 1.0, jnp.float32)

    def body(i, carry):
        idx = dstv[pl.ds(i * 16, 16)]
        plsc.addupdate_scatter(degv, [idx], ones)
        return carry

    lax.fori_loop(0, _EPW // 16, body, 0)
    pltpu.sync_copy(degv, degp.at[wid])


def _scale_body(degp_ref, dis_ref, inv_ref):
    deg = jnp.sum(degp_ref[...], axis=0) + 1.0
    inv = 1.0 / deg
    inv_ref[...] = inv
    dis_ref[...] = lax.rsqrt(deg)


_scale_call = pl.pallas_call(
    _scale_body,
    out_shape=(
        jax.ShapeDtypeStruct((_NPAD,), jnp.float32),
        jax.ShapeDtypeStruct((_NPAD,), jnp.float32),
    ),
)


def _proj_body(xin_ref, wp_ref, bp_ref, wg_ref, dis_ref, x_ref, xw_ref, y_ref):
    x = (
        jnp.dot(xin_ref[...], wp_ref[...], precision=lax.Precision.DEFAULT,
                preferred_element_type=jnp.float32)
        + bp_ref[...]
    )
    xw = jnp.dot(x, wg_ref[...], precision=lax.Precision.DEFAULT,
                 preferred_element_type=jnp.float32)
    x_ref[...] = x
    xw_ref[...] = xw
    y_ref[...] = xw * dis_ref[...]


_proj_call = pl.pallas_call(
    _proj_body,
    out_shape=(
        jax.ShapeDtypeStruct((_N, _D), jnp.float32),
        jax.ShapeDtypeStruct((_N, _D), jnp.float32),
        jax.ShapeDtypeStruct((_N, _D), jnp.float32),
    ),
)


@functools.partial(
    pl.kernel,
    out_type=jax.ShapeDtypeStruct((2, _N, _D), jnp.float32),
    mesh=_mesh,
    scratch_types=[
        pltpu.VMEM((_P, _K), jnp.int32),
        pltpu.VMEM((_P, _K), jnp.int32),
    ]
    + [pltpu.VMEM((_K, _D), jnp.float32)] * _NB
    + [pltpu.VMEM_SHARED((_NPAD, _D), jnp.float32)]
    + [pltpu.SemaphoreType.DMA] * (2 * _NB),
)
def _scat_kernel(y, srct, dstt, z2d, accp, src_v, dst_v, *bufs):
    rows = list(bufs[:_NB])
    acc = bufs[_NB]
    gs = list(bufs[_NB + 1:_NB + 1 + _NB])
    ss = list(bufs[_NB + 1 + _NB:])
    cid = lax.axis_index("c")
    sid = lax.axis_index("s")
    pltpu.sync_copy(z2d, acc.at[pl.ds(sid * 640, 640)])
    plsc.subcore_barrier()

    chunk_base = jnp.where(cid == 0, sid * _CH0, 16 * _CH0 + sid * _CH1)
    nblk = jnp.where(cid == 0, _CH0 // _P, _CH1 // _P)

    for p in range(_CH0 // _P):

        @pl.when(p < nblk)
        def _block():
            pltpu.sync_copy(srct.at[pl.ds(chunk_base + p * _P, _P)], src_v)
            pltpu.sync_copy(dstt.at[pl.ds(chunk_base + p * _P, _P)], dst_v)
            for j0 in range(_NB - 1):
                pltpu.async_copy(y.at[src_v.at[j0]], rows[j0], gs[j0])

            def body(i, carry):
                for b in range(_NB):
                    j = i * _NB + b
                    ob = (b - 1) % _NB
                    pltpu.make_async_copy(y.at[src_v.at[j]], rows[b],
                                          gs[b]).wait()
                    pltpu.async_copy(rows[b], acc.at[dst_v.at[j]], ss[b],
                                     add=True)

                    @pl.when(j >= 1)
                    def _wait_prev_scatter():
                        pltpu.make_async_copy(
                            rows[ob], acc.at[dst_v.at[j - 1]], ss[ob]).wait()

                    @pl.when(j + _NB - 1 < _P)
                    def _start_next_gather():
                        pltpu.async_copy(
                            y.at[src_v.at[j + _NB - 1]], rows[ob], gs[ob])
                return carry

            lax.fori_loop(0, _P // _NB, body, 0)
            pltpu.make_async_copy(
                rows[_NB - 1], acc.at[dst_v.at[_P - 1]], ss[_NB - 1]).wait()

    plsc.subcore_barrier()
    pltpu.sync_copy(
        acc.at[pl.ds(sid * 624, 624)],
        accp.at[cid, pl.ds(sid * 624, 624)],
    )

    @pl.when(sid == 15)
    def _tail():
        pltpu.sync_copy(
            acc.at[pl.ds(9984, 16)],
            accp.at[cid, pl.ds(9984, 16)],
        )


_RB = 1000


def _epi_body(accp_ref, xw_ref, x_ref, dis_ref, inv_ref, bg_ref, gam_ref,
              bet_ref, qm_ref, g_ref, logit_ref):
    agg = accp_ref[0] + accp_ref[1]
    out = agg * dis_ref[...] + xw_ref[...] * inv_ref[...] + bg_ref[...]
    h = out * 0.5 * (1.0 + lax.erf(out * 0.7071067811865476)) + x_ref[...]
    mu = jnp.mean(h, axis=1, keepdims=True)
    var = jnp.mean((h - mu) ** 2, axis=1, keepdims=True)
    g = (h - mu) * lax.rsqrt(var + 1e-5) * gam_ref[...] + bet_ref[...]
    g_ref[...] = g
    logit_ref[...] = jnp.dot(g, qm_ref[...], precision=lax.Precision.DEFAULT,
                             preferred_element_type=jnp.float32) * 10.0


_epi_call = pl.pallas_call(
    _epi_body,
    grid=(_N // _RB,),
    in_specs=[
        pl.BlockSpec((2, _RB, _D), lambda i: (0, i, 0)),
        pl.BlockSpec((_RB, _D), lambda i: (i, 0)),
        pl.BlockSpec((_RB, _D), lambda i: (i, 0)),
        pl.BlockSpec((_RB, 1), lambda i: (i, 0)),
        pl.BlockSpec((_RB, 1), lambda i: (i, 0)),
        pl.BlockSpec((1, _D), lambda i: (0, 0)),
        pl.BlockSpec((1, _D), lambda i: (0, 0)),
        pl.BlockSpec((1, _D), lambda i: (0, 0)),
        pl.BlockSpec((_D, _MH), lambda i: (0, 0)),
    ],
    out_specs=(
        pl.BlockSpec((_RB, _D), lambda i: (i, 0)),
        pl.BlockSpec((_RB, _MH), lambda i: (i, 0)),
    ),
    out_shape=(
        jax.ShapeDtypeStruct((_N, _D), jnp.float32),
        jax.ShapeDtypeStruct((_N, _MH), jnp.float32),
    ),
)


def _soft_body(logit_ref, soft_ref):
    logits = logit_ref[...]
    bid = lax.broadcasted_iota(jnp.int32, (_N, _MH), 0) // (_N // _B)
    msel = jnp.zeros_like(logits)
    for b in range(_B):
        mask = bid == b
        mb = jnp.max(jnp.where(mask, logits, -1e30), axis=0, keepdims=True)
        msel = jnp.where(mask, mb, msel)
    e = jnp.exp(logits - msel)
    ssel = jnp.ones_like(e)
    for b in range(_B):
        mask = bid == b
        sb = jnp.sum(jnp.where(mask, e, 0.0), axis=0, keepdims=True)
        ssel = jnp.where(mask, sb, ssel)
    soft_ref[...] = e / ssel


_soft_call = pl.pallas_call(
    _soft_body,
    out_shape=jax.ShapeDtypeStruct((_N, _MH), jnp.float32),
)


def kernel(img, all_edge, Wp, bp, diff_query, Wg, bg, ln_gamma, ln_beta):
    xin = jnp.transpose(img, (0, 2, 3, 1)).reshape(-1, _C)
    src = all_edge[0]
    dst = all_edge[1]
    pad = _EPAD - _E
    srcp = jnp.concatenate([src, jnp.zeros((pad,), jnp.int32)])
    dstp = jnp.concatenate([dst, jnp.full((pad,), _N, jnp.int32)])
    src_tab = srcp.reshape(_TOTCH, _K)
    dst_tab = dstp.reshape(_TOTCH, _K)
    dst_flat = dstp.reshape(_NW, _EPW)
    z1d = jnp.zeros((_NPAD,), jnp.float32)
    z2d = jnp.zeros((640, _D), jnp.float32)
    dq = diff_query.reshape(_MH, _HD)
    qmat = (dq[:, :, None] * jnp.eye(_MH, dtype=jnp.float32)[:, None, :])
    qmat = qmat.reshape(_D, _MH)

    degp = _deg_kernel(dst_flat, z1d)
    dis, inv = _scale_call(degp)
    dis_col = dis[:_N].reshape(_N, 1)
    inv_col = inv[:_N].reshape(_N, 1)
    x, xw, y = _proj_call(xin, Wp, bp.reshape(1, _D), Wg, dis_col)
    y2 = jnp.concatenate([y, y], axis=0)
    core1_off = jnp.where(jnp.arange(_EPAD, dtype=jnp.int32)
                          < 16 * _CH0 * _K, 0, _N).astype(jnp.int32)
    src_tab2 = (srcp + core1_off).reshape(_TOTCH, _K)
    accp = _scat_kernel(y2, src_tab2, dst_tab, z2d)
    g, logits = _epi_call(accp, xw, x, dis_col, inv_col, bg.reshape(1, _D),
                          ln_gamma.reshape(1, _D), ln_beta.reshape(1, _D),
                          qmat)
    soft = _soft_call(logits)
    adj = jnp.transpose(soft.reshape(_B, _N // _B, _MH), (0, 2, 1))
    adj = adj.reshape(_B * _MH, 1, _N // _B)
    return g, adj


# speedup vs baseline: 2.9525x; 2.9525x over previous
import dataclasses
import functools

import jax
import jax.numpy as jnp
from jax import lax
from jax.experimental import pallas as pl
from jax.experimental.pallas import tpu as pltpu
from jax.experimental.pallas import tpu_sc as plsc

_B, _C, _H, _W = 4, 128, 50, 50
_D = 128
_MH, _HD = 8, 16
_N = _B * _H * _W
_E = 320000
_NW = 32
_K = 64
_P = 32
_NB = 4
_CH0 = 96
_CH1 = 224
_TOTCH = 16 * (_CH0 + _CH1)
_EPW = 10240
_EPAD = _TOTCH * _K
_NPAD = 10240

_mesh = plsc.VectorSubcoreMesh(core_axis_name="c", subcore_axis_name="s")

_sc_params = pltpu.CompilerParams()
if "needs_layout_passes" in pltpu.CompilerParams.__dataclass_fields__:
    _sc_params = dataclasses.replace(_sc_params, needs_layout_passes=False)


@functools.partial(
    pl.kernel,
    out_type=jax.ShapeDtypeStruct((_NW, _NPAD), jnp.float32),
    mesh=_mesh,
    compiler_params=_sc_params,
    scratch_types=[
        pltpu.VMEM((_EPW,), jnp.int32),
        pltpu.VMEM((_NPAD,), jnp.float32),
    ],
)
def _deg_kernel(dst_flat, z1d, degp, dstv, degv):
    cid = lax.axis_index("c")
    sid = lax.axis_index("s")
    wid = sid * 2 + cid
    pltpu.sync_copy(z1d, degv)
    pltpu.sync_copy(dst_flat.at[wid], dstv)
    ones = jnp.full((16,), 1.0, jnp.float32)

    def body(i, carry):
        idx = dstv[pl.ds(i * 16, 16)]
        plsc.addupdate_scatter(degv, [idx], ones)
        return carry

    lax.fori_loop(0, _EPW // 16, body, 0)
    pltpu.sync_copy(degv, degp.at[wid])


def _scale_body(degp_ref, dis_ref, inv_ref):
    deg = jnp.sum(degp_ref[...], axis=0) + 1.0
    inv = 1.0 / deg
    inv_ref[...] = inv
    dis_ref[...] = lax.rsqrt(deg)


_scale_call = pl.pallas_call(
    _scale_body,
    out_shape=(
        jax.ShapeDtypeStruct((_NPAD,), jnp.float32),
        jax.ShapeDtypeStruct((_NPAD,), jnp.float32),
    ),
)


def _proj_body(xin_ref, wp_ref, bp_ref, wg_ref, dis_ref, x_ref, xw_ref, y_ref):
    x = (
        jnp.dot(xin_ref[...], wp_ref[...], precision=lax.Precision.DEFAULT,
                preferred_element_type=jnp.float32)
        + bp_ref[...]
    )
    xw = jnp.dot(x, wg_ref[...], precision=lax.Precision.DEFAULT,
                 preferred_element_type=jnp.float32)
    x_ref[...] = x
    xw_ref[...] = xw
    y_ref[...] = xw * dis_ref[...]


_proj_call = pl.pallas_call(
    _proj_body,
    out_shape=(
        jax.ShapeDtypeStruct((_N, _D), jnp.float32),
        jax.ShapeDtypeStruct((_N, _D), jnp.float32),
        jax.ShapeDtypeStruct((_N, _D), jnp.float32),
    ),
)


@functools.partial(
    pl.kernel,
    out_type=jax.ShapeDtypeStruct((2, _N, _D), jnp.float32),
    mesh=_mesh,
    scratch_types=[
        pltpu.VMEM((_P, _K), jnp.int32),
        pltpu.VMEM((_P, _K), jnp.int32),
    ]
    + [pltpu.VMEM((_K, _D), jnp.float32)] * _NB
    + [pltpu.VMEM_SHARED((_NPAD, _D), jnp.float32)]
    + [pltpu.SemaphoreType.DMA] * (2 * _NB),
)
def _scat_kernel(y, srct, dstt, z2d, accp, src_v, dst_v, *bufs):
    rows = list(bufs[:_NB])
    acc = bufs[_NB]
    gs = list(bufs[_NB + 1:_NB + 1 + _NB])
    ss = list(bufs[_NB + 1 + _NB:])
    cid = lax.axis_index("c")
    sid = lax.axis_index("s")
    pltpu.sync_copy(z2d, acc.at[pl.ds(sid * 640, 640)])
    plsc.subcore_barrier()

    chunk_base = jnp.where(cid == 0, sid * _CH0, 16 * _CH0 + sid * _CH1)
    nblk = jnp.where(cid == 0, _CH0 // _P, _CH1 // _P)

    for p in range(_CH0 // _P):

        @pl.when(p < nblk)
        def _block():
            pltpu.sync_copy(srct.at[pl.ds(chunk_base + p * _P, _P)], src_v)
            pltpu.sync_copy(dstt.at[pl.ds(chunk_base + p * _P, _P)], dst_v)
            for j0 in range(_NB - 1):
                pltpu.async_copy(y.at[src_v.at[j0]], rows[j0], gs[j0])

            def body(i, carry):
                for b in range(_NB):
                    j = i * _NB + b
                    ob = (b - 1) % _NB
                    pltpu.make_async_copy(y.at[src_v.at[j]], rows[b],
                                          gs[b]).wait()
                    pltpu.async_copy(rows[b], acc.at[dst_v.at[j]], ss[b],
                                     add=True)

                    @pl.when(j >= 1)
                    def _wait_prev_scatter():
                        pltpu.make_async_copy(
                            rows[ob], acc.at[dst_v.at[j - 1]], ss[ob]).wait()

                    @pl.when(j + _NB - 1 < _P)
                    def _start_next_gather():
                        pltpu.async_copy(
                            y.at[src_v.at[j + _NB - 1]], rows[ob], gs[ob])
                return carry

            lax.fori_loop(0, _P // _NB, body, 0)
            pltpu.make_async_copy(
                rows[_NB - 1], acc.at[dst_v.at[_P - 1]], ss[_NB - 1]).wait()

    plsc.subcore_barrier()
    pltpu.sync_copy(
        acc.at[pl.ds(sid * 624, 624)],
        accp.at[cid, pl.ds(sid * 624, 624)],
    )

    @pl.when(sid == 15)
    def _tail():
        pltpu.sync_copy(
            acc.at[pl.ds(9984, 16)],
            accp.at[cid, pl.ds(9984, 16)],
        )


_RB = 1000


def _epi_body(accp_ref, xw_ref, x_ref, dis_ref, inv_ref, bg_ref, gam_ref,
              bet_ref, qm_ref, g_ref, logit_ref):
    agg = accp_ref[0] + accp_ref[1]
    out = agg * dis_ref[...] + xw_ref[...] * inv_ref[...] + bg_ref[...]
    h = out * 0.5 * (1.0 + lax.erf(out * 0.7071067811865476)) + x_ref[...]
    mu = jnp.mean(h, axis=1, keepdims=True)
    var = jnp.mean((h - mu) ** 2, axis=1, keepdims=True)
    g = (h - mu) * lax.rsqrt(var + 1e-5) * gam_ref[...] + bet_ref[...]
    g_ref[...] = g
    logit_ref[...] = jnp.dot(g, qm_ref[...], precision=lax.Precision.DEFAULT,
                             preferred_element_type=jnp.float32) * 10.0


_epi_call = pl.pallas_call(
    _epi_body,
    grid=(_N // _RB,),
    in_specs=[
        pl.BlockSpec((2, _RB, _D), lambda i: (0, i, 0)),
        pl.BlockSpec((_RB, _D), lambda i: (i, 0)),
        pl.BlockSpec((_RB, _D), lambda i: (i, 0)),
        pl.BlockSpec((_RB, 1), lambda i: (i, 0)),
        pl.BlockSpec((_RB, 1), lambda i: (i, 0)),
        pl.BlockSpec((1, _D), lambda i: (0, 0)),
        pl.BlockSpec((1, _D), lambda i: (0, 0)),
        pl.BlockSpec((1, _D), lambda i: (0, 0)),
        pl.BlockSpec((_D, _MH), lambda i: (0, 0)),
    ],
    out_specs=(
        pl.BlockSpec((_RB, _D), lambda i: (i, 0)),
        pl.BlockSpec((_RB, _MH), lambda i: (i, 0)),
    ),
    out_shape=(
        jax.ShapeDtypeStruct((_N, _D), jnp.float32),
        jax.ShapeDtypeStruct((_N, _MH), jnp.float32),
    ),
)


def _soft_body(logit_ref, soft_ref):
    logits = logit_ref[...]
    bid = lax.broadcasted_iota(jnp.int32, (_N, _MH), 0) // (_N // _B)
    msel = jnp.zeros_like(logits)
    for b in range(_B):
        mask = bid == b
        mb = jnp.max(jnp.where(mask, logits, -1e30), axis=0, keepdims=True)
        msel = jnp.where(mask, mb, msel)
    e = jnp.exp(logits - msel)
    ssel = jnp.ones_like(e)
    for b in range(_B):
        mask = bid == b
        sb = jnp.sum(jnp.where(mask, e, 0.0), axis=0, keepdims=True)
        ssel = jnp.where(mask, sb, ssel)
    soft_ref[...] = e / ssel


_soft_call = pl.pallas_call(
    _soft_body,
    out_shape=jax.ShapeDtypeStruct((_N, _MH), jnp.float32),
)


def kernel(img, all_edge, Wp, bp, diff_query, Wg, bg, ln_gamma, ln_beta):
    xin = jnp.transpose(img, (0, 2, 3, 1)).reshape(-1, _C)
    src = all_edge[0]
    dst = all_edge[1]
    pad = _EPAD - _E
    srcp = jnp.concatenate([src, jnp.zeros((pad,), jnp.int32)])
    dstp = jnp.concatenate([dst, jnp.full((pad,), _N, jnp.int32)])
    src_tab = srcp.reshape(_TOTCH, _K)
    dst_tab = dstp.reshape(_TOTCH, _K)
    dst_flat = dstp.reshape(_NW, _EPW)
    z1d = jnp.zeros((_NPAD,), jnp.float32)
    z2d = jnp.zeros((640, _D), jnp.float32)
    dq = diff_query.reshape(_MH, _HD)
    qmat = (dq[:, :, None] * jnp.eye(_MH, dtype=jnp.float32)[:, None, :])
    qmat = qmat.reshape(_D, _MH)

    degp = _deg_kernel(dst_flat, z1d)
    dis, inv = _scale_call(degp)
    dis_col = dis[:_N].reshape(_N, 1)
    inv_col = inv[:_N].reshape(_N, 1)
    x, xw, y = _proj_call(xin, Wp, bp.reshape(1, _D), Wg, dis_col)
    accp = _scat_kernel(y, src_tab, dst_tab, z2d)
    g, logits = _epi_call(accp, xw, x, dis_col, inv_col, bg.reshape(1, _D),
                          ln_gamma.reshape(1, _D), ln_beta.reshape(1, _D),
                          qmat)
    soft = _soft_call(logits)
    adj = jnp.transpose(soft.reshape(_B, _N // _B, _MH), (0, 2, 1))
    adj = adj.reshape(_B * _MH, 1, _N // _B)
    return g, adj
